# baseline (speedup 1.0000x reference)
import functools

import jax
import jax.numpy as jnp
from jax import lax
from jax.experimental import pallas as pl
from jax.experimental.pallas import tpu as pltpu
from jax.experimental.pallas import tpu_sc as plsc

_C = 8
_RING = 3
_UNROLL = 2


def _make_sc_kernel(B, T, D):
    info = plsc.get_sparse_core_info()
    NC, NS, L = info.num_cores, info.num_subcores, info.num_lanes
    NW = NC * NS
    rows_per_w = T // NW
    n_chunks = rows_per_w // _C
    n_units = n_chunks * B
    mesh = plsc.VectorSubcoreMesh(core_axis_name="c", subcore_axis_name="s")

    @functools.partial(
        pl.kernel,
        mesh=mesh,
        out_type=jax.ShapeDtypeStruct((B, T, D), jnp.float32),
        scratch_types=(
            [pltpu.VMEM((_C, D), jnp.float32) for _ in range(2 * _RING)]
            + [pltpu.SemaphoreType.DMA for _ in range(3 * _RING)]
        ),
    )
    def k(x_hbm, e_hbm, o_hbm, xb0, xb1, xb2, eb0, eb1, eb2,
          sl0, sl1, sl2, ss0, ss1, ss2, se0, se1, se2):
        xb = (xb0, xb1, xb2)
        eb = (eb0, eb1, eb2)
        sld = (sl0, sl1, sl2)
        sst = (ss0, ss1, ss2)
        sle = (se0, se1, se2)
        wid = lax.axis_index("s") * NC + lax.axis_index("c")
        t0 = wid * rows_per_w

        def fire_x(u, r):
            ci, b = u // B, u % B
            pltpu.async_copy(
                x_hbm.at[b, pl.ds(t0 + ci * _C, _C)], xb[r], sld[r])

        def fire_e(ci, r):
            pltpu.async_copy(
                e_hbm.at[pl.ds(t0 + ci * _C, _C)], eb[r], sle[r])

        def drain_x(r):
            pltpu.make_async_copy(
                x_hbm.at[0, pl.ds(0, _C)], xb[r], sld[r]).wait()

        def drain_e(r):
            pltpu.make_async_copy(
                e_hbm.at[pl.ds(0, _C)], eb[r], sle[r]).wait()

        def fire_store(u, r):
            ci, b = u // B, u % B
            pltpu.async_copy(
                xb[r], o_hbm.at[b, pl.ds(t0 + ci * _C, _C)], sst[r])

        def drain_store(r):
            pltpu.make_async_copy(
                xb[r], o_hbm.at[0, pl.ds(0, _C)], sst[r]).wait()

        def compute(jx, je):
            xr, er = xb[jx], eb[je]

            @plsc.parallel_loop(0, D, step=L, unroll=_UNROLL)
            def _(col):
                for q in range(_C):
                    xr[q, pl.ds(col, L)] = (
                        xr[q, pl.ds(col, L)] + er[q, pl.ds(col, L)])

        def unit(u, jx, je, b, drain_st, fire_ld, fire_emb):
            nxt = (jx + 1) % _RING
            if drain_st:
                drain_store(nxt)
            if fire_ld:
                fire_x(u + 1, nxt)
            if b == 0:
                if fire_emb:
                    fire_e(u // B + 1, (je + 1) % _RING)
                drain_e(je)
            drain_x(jx)
            compute(jx, je)
            fire_store(u, jx)

        fire_e(0, 0)
        fire_x(0, 0)
        for j in range(3 * B):
            ci, b = j // B, j % B
            unit(j, j % _RING, ci % _RING, b,
                 drain_st=(j >= 2), fire_ld=True,
                 fire_emb=(ci + 1 < n_chunks))

        def body(kk, _):
            u0 = kk * 3 * B
            for j in range(3 * B):
                ci_off, b = j // B, j % B
                unit(u0 + j, j % _RING, ci_off % _RING, b,
                     drain_st=True, fire_ld=True, fire_emb=True)
            return 0

        lax.fori_loop(1, n_chunks // 3, body, 0)

        for j in range(B):
            u = (n_chunks - 1) * B + j
            unit(u, u % _RING, (n_chunks - 1) % _RING, j,
                 drain_st=True, fire_ld=(j + 1 < B), fire_emb=False)

        drain_store((n_units - 2) % _RING)
        drain_store((n_units - 1) % _RING)

    return k


def kernel(x, emb):
    B, T, D = x.shape
    k = _make_sc_kernel(B, T, D)
    return k(x, emb[:T])
